# baseline (speedup 1.0000x reference)
import functools

import jax
import jax.numpy as jnp
from jax import lax
from jax.experimental import pallas as pl
from jax.experimental.pallas import tpu as pltpu
from jax.experimental.pallas import tpu_sc as plsc

_INFO = plsc.get_sparse_core_info()
_NC = _INFO.num_cores
_NS = _INFO.num_subcores
_NW = _NC * _NS

_B = 16384
_D = 128
_B_PER_W = _B // _NW
_CHUNK = 128
_NCHUNK = _B_PER_W // _CHUNK
_HALF_TILES = _NS // 2
_STAGE_ROWS = _HALF_TILES * _B_PER_W


@functools.partial(
    pl.kernel,
    mesh=plsc.VectorSubcoreMesh(core_axis_name="c", subcore_axis_name="s"),
    out_type=jax.ShapeDtypeStruct((_B, _D), jnp.float32),
    scratch_types=[
        pltpu.VMEM((_NCHUNK, _CHUNK), jnp.int32),
        pltpu.VMEM((_B_PER_W, _D), jnp.float32),
        pltpu.VMEM_SHARED((_STAGE_ROWS, _D), jnp.float32),
        pltpu.SemaphoreType.DMA,
    ],
)
def _gather_rows(table_hbm, idx_hbm, out_hbm, idx_v, rows_v, shared, sem):
    c = lax.axis_index("c")
    s = lax.axis_index("s")
    wid = c * _NS + s
    base = wid * _B_PER_W
    pltpu.sync_copy(idx_hbm.at[pl.ds(wid * _NCHUNK, _NCHUNK)], idx_v)
    gathers = [
        pltpu.async_copy(
            table_hbm.at[idx_v.at[j]],
            rows_v.at[pl.ds(j * _CHUNK, _CHUNK)],
            sem,
        )
        for j in range(_NCHUNK)
    ]
    for g in gathers:
        g.wait()

    @pl.when(s < _HALF_TILES)
    def _direct():
        pltpu.sync_copy(rows_v, out_hbm.at[pl.ds(base, _B_PER_W)])

    @pl.when(s >= _HALF_TILES)
    def _stage():
        pltpu.sync_copy(
            rows_v, shared.at[pl.ds((s - _HALF_TILES) * _B_PER_W, _B_PER_W)]
        )

    plsc.subcore_barrier()

    @pl.when(s == _HALF_TILES)
    def _flush():
        pltpu.sync_copy(
            shared,
            out_hbm.at[pl.ds(c * _NS * _B_PER_W + _STAGE_ROWS, _STAGE_ROWS)],
        )


@jax.jit
def kernel(input_tensor, indices):
    idx2d = indices.reshape(_NW * _NCHUNK, _CHUNK).astype(jnp.int32)
    return _gather_rows(input_tensor, idx2d)


# speedup vs baseline: 1.1404x; 1.1404x over previous
import functools

import jax
import jax.numpy as jnp
from jax import lax
from jax.experimental import pallas as pl
from jax.experimental.pallas import tpu as pltpu
from jax.experimental.pallas import tpu_sc as plsc

_INFO = plsc.get_sparse_core_info()
_NC = _INFO.num_cores
_NS = _INFO.num_subcores
_NW = _NC * _NS

_B = 16384
_D = 128
_B_PER_W = _B // _NW
_CHUNK = 128
_NCHUNK = _B_PER_W // _CHUNK


@functools.partial(
    pl.kernel,
    mesh=plsc.VectorSubcoreMesh(core_axis_name="c", subcore_axis_name="s"),
    out_type=jax.ShapeDtypeStruct((_B, _D), jnp.float32),
    scratch_types=[
        pltpu.VMEM((_NCHUNK, _CHUNK), jnp.int32),
        pltpu.VMEM((_B_PER_W, _D), jnp.float32),
    ]
    + [pltpu.SemaphoreType.DMA] * 3,
)
def _gather_rows(table_hbm, idx_hbm, out_hbm, idx_v, rows_v, sem_a, sem_b, sem_w):
    half = _B_PER_W // 2
    wid = lax.axis_index("s") * _NC + lax.axis_index("c")
    base = wid * _B_PER_W
    pltpu.sync_copy(idx_hbm.at[pl.ds(wid * _NCHUNK, _NCHUNK)], idx_v)
    gathers = [
        pltpu.async_copy(
            table_hbm.at[idx_v.at[j]],
            rows_v.at[pl.ds(j * _CHUNK, _CHUNK)],
            sem_a if j < _NCHUNK // 2 else sem_b,
        )
        for j in range(_NCHUNK)
    ]
    for c in gathers[: _NCHUNK // 2]:
        c.wait()
    w = pltpu.async_copy(
        rows_v.at[pl.ds(0, half)], out_hbm.at[pl.ds(base, half)], sem_w
    )
    for c in gathers[_NCHUNK // 2 :]:
        c.wait()
    pltpu.sync_copy(
        rows_v.at[pl.ds(half, half)], out_hbm.at[pl.ds(base + half, half)]
    )
    w.wait()


@jax.jit
def kernel(input_tensor, indices):
    idx2d = indices.reshape(_NW * _NCHUNK, _CHUNK).astype(jnp.int32)
    return _gather_rows(input_tensor, idx2d)
